# Initial kernel scaffold; baseline (speedup 1.0000x reference)
import jax
import jax.numpy as jnp
from jax.experimental import pallas as pl


def kernel(rel_emb, length):
    raise NotImplementedError("write your pallas kernel here")


# speedup vs baseline: 12.1599x; 12.1599x over previous
import functools

import jax
import jax.numpy as jnp
from jax import lax
from jax.experimental import pallas as pl
from jax.experimental.pallas import tpu as pltpu
from jax.experimental.pallas import tpu_sc as plsc


@functools.lru_cache(maxsize=None)
def _build_sc_kernel(V, D, L, R, C, K):
    NCB = L // C
    NRB = L // R
    W = C + R

    info = plsc.get_sparse_core_info()
    num_workers = info.num_cores * info.num_subcores
    tiles = NRB * NCB
    assert tiles % num_workers == 0
    tiles_per_worker = tiles // num_workers

    mesh = plsc.VectorSubcoreMesh(core_axis_name="c", subcore_axis_name="s")

    @functools.partial(
        pl.kernel,
        out_type=jax.ShapeDtypeStruct((L * L, D), jnp.float32),
        name="toeplitz_gather_sc",
        mesh=mesh,
        scratch_types=[
            pltpu.VMEM((W, D), jnp.float32),
            pltpu.SemaphoreType.DMA,
        ],
    )
    def sc_kernel(table, out, win, sem):
        wid = lax.axis_index("s") * info.num_cores + lax.axis_index("c")

        def tile_body(t, carry):
            tid = wid * tiles_per_worker + t
            rb = tid // NCB
            cb = tid % NCB
            i0 = rb * R
            j0 = cb * C
            base = pl.multiple_of((L - 1) + j0 - i0 - (R - 1), 8)
            pltpu.sync_copy(table.at[pl.ds(base, W)], win)

            def fire(r, c):
                pltpu.async_copy(
                    win.at[pl.ds(R - 1 - r, C)],
                    out.at[pl.ds(pl.multiple_of((i0 + r) * L + j0, 8), C)],
                    sem,
                )
                return c

            def wait_one(r, c):
                pltpu.make_async_copy(
                    win.at[pl.ds(0, C)], out.at[pl.ds(j0, C)], sem
                ).wait()
                return c

            def steady(r, c):
                c = wait_one(r, c)
                return fire(r, c)

            carry = lax.fori_loop(0, K, fire, carry)
            carry = lax.fori_loop(K, R, steady, carry)
            carry = lax.fori_loop(0, K, wait_one, carry)
            return carry

        lax.fori_loop(0, tiles_per_worker, tile_body, 0)

    return sc_kernel


def kernel(rel_emb, length):
    V, D = rel_emb.shape
    L = (V + 1) // 2
    table = jnp.concatenate([rel_emb, jnp.zeros((1, D), rel_emb.dtype)], axis=0)
    out2d = _build_sc_kernel(V + 1, D, L, 256, 512, 8)(table)
    return out2d.reshape(L, L, D)


# speedup vs baseline: 12.1715x; 1.0010x over previous
import functools

import jax
import jax.numpy as jnp
from jax import lax
from jax.experimental import pallas as pl
from jax.experimental.pallas import tpu as pltpu
from jax.experimental.pallas import tpu_sc as plsc


@functools.lru_cache(maxsize=None)
def _build_sc_kernel(V, D, L, R, C, K):
    NCB = L // C
    NRB = L // R
    W = C + R

    info = plsc.get_sparse_core_info()
    num_workers = info.num_cores * info.num_subcores
    tiles = NRB * NCB
    assert tiles % num_workers == 0
    tiles_per_worker = tiles // num_workers

    mesh = plsc.VectorSubcoreMesh(core_axis_name="c", subcore_axis_name="s")

    @functools.partial(
        pl.kernel,
        out_type=jax.ShapeDtypeStruct((L * L, D), jnp.float32),
        name="toeplitz_gather_sc",
        mesh=mesh,
        scratch_types=[
            pltpu.VMEM((W, D), jnp.float32),
            pltpu.SemaphoreType.DMA,
        ],
    )
    def sc_kernel(table, out, win, sem):
        wid = lax.axis_index("s") * info.num_cores + lax.axis_index("c")

        def tile_body(t, carry):
            tid = wid * tiles_per_worker + t
            rb = tid // NCB
            cb = tid % NCB
            i0 = rb * R
            j0 = cb * C
            base = pl.multiple_of((L - 1) + j0 - i0 - (R - 1), 8)
            pltpu.sync_copy(table.at[pl.ds(base, W)], win)

            def fire(r, c):
                pltpu.async_copy(
                    win.at[pl.ds(R - 1 - r, C)],
                    out.at[pl.ds(pl.multiple_of((i0 + r) * L + j0, 8), C)],
                    sem,
                )
                return c

            def wait_one(r, c):
                pltpu.make_async_copy(
                    win.at[pl.ds(0, C)], out.at[pl.ds(j0, C)], sem
                ).wait()
                return c

            def steady(r, c):
                c = wait_one(r, c)
                return fire(r, c)

            carry = lax.fori_loop(0, K, fire, carry)
            carry = lax.fori_loop(K, R, steady, carry)
            carry = lax.fori_loop(0, K, wait_one, carry)
            return carry

        lax.fori_loop(0, tiles_per_worker, tile_body, 0)

    return sc_kernel


def kernel(rel_emb, length):
    V, D = rel_emb.shape
    L = (V + 1) // 2
    table = jnp.concatenate([rel_emb, jnp.zeros((1, D), rel_emb.dtype)], axis=0)
    out2d = _build_sc_kernel(V + 1, D, L, 256, 512, 16)(table)
    return out2d.reshape(L, L, D)
